# baseline (speedup 1.0000x reference)
import functools

import jax
import jax.numpy as jnp
from jax import lax
from jax.experimental import pallas as pl
from jax.experimental.pallas import tpu as pltpu
from jax.experimental.pallas import tpu_sc as plsc

_NC = 2
_NS = 16
_NW = _NC * _NS
_L = 16
_NB = 4
_GP = 130


@functools.lru_cache(maxsize=None)
def _make(V, D, H, B):
    assert D == 32 and V % 4 == 0
    bpw = B // _NW
    nslab = H * (bpw // 128)
    assert nslab % _NB == 0
    h_lo = (H // 8) * 8
    mesh = plsc.VectorSubcoreMesh(core_axis_name="c", subcore_axis_name="s")

    @functools.partial(
        pl.kernel,
        mesh=mesh,
        out_type=jax.ShapeDtypeStruct((H, D, B), jnp.float32),
        scratch_types=[
            pltpu.VMEM((H, bpw), jnp.int32),
            pltpu.VMEM((_NB, 128), jnp.int32),
            pltpu.VMEM((_NB, 128), jnp.int32),
            pltpu.VMEM((_NB, 128, 128), jnp.float32),
            pltpu.VMEM((_NB, D, 128), jnp.float32),
            [pltpu.SemaphoreType.DMA] * _NB,
            [pltpu.SemaphoreType.DMA] * _NB,
        ],
        compiler_params=pltpu.CompilerParams(
            use_tc_tiling_on_sc=True, needs_layout_passes=False),
    )
    def k(tbl_hbm, xt_hbm, out_hbm, idx_all, idx_q, col_q, gbuf, ostage,
          gsem, ssem):
        wid = lax.axis_index("s") * _NC + lax.axis_index("c")
        col0 = wid * bpw
        pltpu.sync_copy(xt_hbm.at[pl.ds(0, h_lo), pl.ds(col0, bpw)],
                        idx_all.at[pl.ds(0, h_lo)])
        pltpu.sync_copy(xt_hbm.at[pl.ds(h_lo, H - h_lo), pl.ds(col0, bpw)],
                        idx_all.at[pl.ds(h_lo, H - h_lo)])

        def prep_and_fire(s, m):
            h = s % H
            bbl = s // H
            for kk in range(8):
                v = idx_all[h, pl.ds(bbl * 128 + kk * _L, _L)]
                idx_q[m, pl.ds(kk * _L, _L)] = lax.shift_right_logical(v, 2)
                col_q[m, pl.ds(kk * _L, _L)] = lax.bitwise_and(v, 3) * D
            pltpu.async_copy(tbl_hbm.at[idx_q.at[m]],
                             gbuf.at[m], gsem[m])

        rowv = [jnp.arange(_L, dtype=jnp.int32) + kk * _L for kk in range(8)]

        for m in range(_NB - 1):
            prep_and_fire(m, m)

        @pl.loop(0, nslab, step=_NB)
        def _(si):
            for m in range(_NB):
                s = si + m

                @pl.when(s + _NB - 1 < nslab)
                def _():
                    prep_and_fire(s + _NB - 1, (m + _NB - 1) % _NB)

                pltpu.make_async_copy(
                    tbl_hbm.at[pl.ds(0, 128)],
                    gbuf.at[m], gsem[m]).wait()

                @pl.when(s >= _NB)
                def _():
                    pltpu.make_async_copy(
                        ostage.at[m],
                        out_hbm.at[0, pl.ds(0, D), pl.ds(0, 128)],
                        ssem[m]).wait()

                colb = [col_q[m, pl.ds(kk * _L, _L)] for kk in range(8)]

                @pl.loop(0, D, step=2)
                def _(d):
                    for u in range(2):
                        drow = lax.bitwise_and(rowv[0] + (d + u), D - 1)
                        for kk in range(8):
                            val = plsc.load_gather(
                                gbuf.at[m], [rowv[kk], colb[kk] + drow])
                            plsc.store_scatter(
                                ostage.at[m], [drow, rowv[kk]], val)

                h = s % H
                bbl = s // H
                pltpu.async_copy(
                    ostage.at[m],
                    out_hbm.at[h, pl.ds(0, D), pl.ds(col0 + bbl * 128, 128)],
                    ssem[m])

        for m in range(_NB):
            pltpu.make_async_copy(
                ostage.at[m],
                out_hbm.at[0, pl.ds(0, D), pl.ds(0, 128)],
                ssem[m]).wait()

    return k


def kernel(x, table):
    Bx, H = x.shape
    V, D = table.shape
    xt = x.astype(jnp.int32).T
    tbl128 = table.reshape(V // 4, 4 * D)
    out3 = _make(V, D, H, Bx)(tbl128, xt)
    return out3.transpose(2, 0, 1)


# speedup vs baseline: 1.4288x; 1.4288x over previous
import functools

import jax
import jax.numpy as jnp
from jax import lax
from jax.experimental import pallas as pl
from jax.experimental.pallas import tpu as pltpu
from jax.experimental.pallas import tpu_sc as plsc

_NC = 2
_NS = 16
_NW = _NC * _NS
_L = 16
_NB = 4
_GP = 130


@functools.lru_cache(maxsize=None)
def _make(V, D, H, B):
    assert D == 32 and V % 4 == 0
    bpw = B // _NW
    nslab = H * (bpw // 128)
    assert nslab % _NB == 0
    h_lo = (H // 8) * 8
    mesh = plsc.VectorSubcoreMesh(core_axis_name="c", subcore_axis_name="s")

    @functools.partial(
        pl.kernel,
        mesh=mesh,
        out_type=jax.ShapeDtypeStruct((H, D, B), jnp.float32),
        scratch_types=[
            pltpu.VMEM((H, bpw), jnp.int32),
            pltpu.VMEM((_NB, 128), jnp.int32),
            pltpu.VMEM((_NB, 128), jnp.int32),
            pltpu.VMEM((_NB, 128, 128), jnp.float32),
            pltpu.VMEM((_NB, D, 128), jnp.float32),
            [pltpu.SemaphoreType.DMA] * _NB,
            [pltpu.SemaphoreType.DMA] * _NB,
        ],
        compiler_params=pltpu.CompilerParams(
            use_tc_tiling_on_sc=True, needs_layout_passes=False),
    )
    def k(tbl_hbm, xt_hbm, out_hbm, idx_all, idx_q, col_q, gbuf, ostage,
          gsem, ssem):
        wid = lax.axis_index("s") * _NC + lax.axis_index("c")
        col0 = wid * bpw
        pltpu.sync_copy(xt_hbm.at[pl.ds(0, h_lo), pl.ds(col0, bpw)],
                        idx_all.at[pl.ds(0, h_lo)])
        pltpu.sync_copy(xt_hbm.at[pl.ds(h_lo, H - h_lo), pl.ds(col0, bpw)],
                        idx_all.at[pl.ds(h_lo, H - h_lo)])

        def prep_and_fire(s, m):
            h = s % H
            bbl = s // H
            for kk in range(8):
                v = idx_all[h, pl.ds(bbl * 128 + kk * _L, _L)]
                idx_q[m, pl.ds(kk * _L, _L)] = lax.shift_right_logical(v, 2)
                col_q[m, pl.ds(kk * _L, _L)] = lax.bitwise_and(v, 3) * D
            pltpu.async_copy(tbl_hbm.at[idx_q.at[m]],
                             gbuf.at[m], gsem[m])

        rowv = [jnp.arange(_L, dtype=jnp.int32) + kk * _L for kk in range(8)]

        for m in range(_NB - 1):
            prep_and_fire(m, m)

        @pl.loop(0, nslab, step=_NB)
        def _(si):
            for m in range(_NB):
                s = si + m

                @pl.when(s + _NB - 1 < nslab)
                def _():
                    prep_and_fire(s + _NB - 1, (m + _NB - 1) % _NB)

                pltpu.make_async_copy(
                    tbl_hbm.at[pl.ds(0, 128)],
                    gbuf.at[m], gsem[m]).wait()

                @pl.when(s >= _NB)
                def _():
                    pltpu.make_async_copy(
                        ostage.at[m],
                        out_hbm.at[0, pl.ds(0, D), pl.ds(0, 128)],
                        ssem[m]).wait()

                colb = [col_q[m, pl.ds(kk * _L, _L)] for kk in range(8)]

                @pl.loop(0, D, step=2)
                def _(d):
                    for u in range(2):
                        drow = lax.bitwise_and(rowv[0] + (d + u), D - 1)
                        for kk in range(8):
                            val = plsc.load_gather(
                                gbuf.at[m], [rowv[kk], colb[kk] + drow])
                            plsc.store_scatter(
                                ostage.at[m], [drow, rowv[kk]], val)

                h = s % H
                bbl = s // H
                pltpu.async_copy(
                    ostage.at[m],
                    out_hbm.at[h, pl.ds(0, D), pl.ds(col0 + bbl * 128, 128)],
                    ssem[m])

        for m in range(_NB):
            pltpu.make_async_copy(
                ostage.at[m],
                out_hbm.at[0, pl.ds(0, D), pl.ds(0, 128)],
                ssem[m]).wait()

    return k


@functools.lru_cache(maxsize=None)
def _make_detile(V, D):
    nt_full = V // 128
    rem = V - nt_full * 128
    base, extra = divmod(nt_full, _NW)
    mesh = plsc.VectorSubcoreMesh(core_axis_name="c", subcore_axis_name="s")

    @functools.partial(
        pl.kernel,
        mesh=mesh,
        out_type=jax.ShapeDtypeStruct((V // 4, 4 * D), jnp.float32),
        scratch_types=[
            pltpu.VMEM((2, D, 128), jnp.float32),
            pltpu.VMEM((2, D, 128), jnp.float32),
            [pltpu.SemaphoreType.DMA] * 2,
            [pltpu.SemaphoreType.DMA] * 2,
        ],
        compiler_params=pltpu.CompilerParams(
            use_tc_tiling_on_sc=True, needs_layout_passes=False),
    )
    def k(tt_hbm, tail_hbm, out_hbm, sbuf, stage, isem, osem):
        wid = lax.axis_index("s") * _NC + lax.axis_index("c")
        t0 = wid * base + jnp.minimum(wid, extra)
        nt = base + jnp.where(wid < extra, 1, 0)

        iota = jnp.arange(_L, dtype=jnp.int32)
        ccol = [iota + 16 * g for g in range(8)]
        srow = [lax.shift_right_logical(c, 2) for c in ccol]
        scb = [lax.bitwise_and(c, 3) * D for c in ccol]

        def fire_in(t, p):
            pltpu.async_copy(
                tt_hbm.at[pl.ds(0, D), pl.ds((t0 + t) * 128, 128)],
                sbuf.at[p], isem[p])

        def transpose(p, ngrp):
            @pl.loop(0, D)
            def _(d0):
                drow = lax.bitwise_and(iota + d0, D - 1)
                for g in range(ngrp):
                    val = plsc.load_gather(sbuf.at[p], [drow, ccol[g]])
                    plsc.store_scatter(
                        stage.at[p], [srow[g], scb[g] + drow], val)

        fire_in(0, 0)

        @pl.loop(0, nt + (nt % 2), step=2)
        def _(ti):
            for p in (0, 1):
                t = ti + p

                @pl.when(t < nt)
                def _():
                    @pl.when(t + 1 < nt)
                    def _():
                        fire_in(t + 1, 1 - p)

                    pltpu.make_async_copy(
                        tt_hbm.at[pl.ds(0, D), pl.ds(0, 128)],
                        sbuf.at[p], isem[p]).wait()

                    @pl.when(t >= 2)
                    def _():
                        pltpu.make_async_copy(
                            stage.at[p], out_hbm.at[pl.ds(0, D)],
                            osem[p]).wait()

                    transpose(p, 8)
                    pltpu.async_copy(
                        stage.at[p], out_hbm.at[pl.ds((t0 + t) * D, D)],
                        osem[p])

        for p in (0, 1):
            @pl.when(nt >= 2 - p)
            def _():
                pltpu.make_async_copy(
                    stage.at[p], out_hbm.at[pl.ds(0, D)], osem[p]).wait()

        @pl.when(wid == _NW - 1)
        def _():
            pltpu.sync_copy(tail_hbm, out_hbm.at[pl.ds(nt_full * D, rem * D // 128)])

    return k


def kernel(x, table):
    Bx, H = x.shape
    V, D = table.shape
    xt = x.astype(jnp.int32).T
    ttile = table.T
    ntf = (V // 128) * 128
    tail128 = table[ntf:].reshape(-1, 4 * D)
    tbl128 = _make_detile(V, D)(ttile, tail128)
    out3 = _make(V, D, H, Bx)(tbl128, xt)
    return out3.transpose(2, 0, 1)


# speedup vs baseline: 1.4442x; 1.0108x over previous
import functools

import jax
import jax.numpy as jnp
from jax import lax
from jax.experimental import pallas as pl
from jax.experimental.pallas import tpu as pltpu
from jax.experimental.pallas import tpu_sc as plsc

_NC = 2
_NS = 16
_NW = _NC * _NS
_L = 16
_NB = 4
_GP = 130


@functools.lru_cache(maxsize=None)
def _make(V, D, H, B):
    assert D == 32 and V % 4 == 0
    bpw = B // _NW
    nslab = H * (bpw // 128)
    assert nslab % _NB == 0
    h_lo = (H // 8) * 8
    mesh = plsc.VectorSubcoreMesh(core_axis_name="c", subcore_axis_name="s")

    @functools.partial(
        pl.kernel,
        mesh=mesh,
        out_type=jax.ShapeDtypeStruct((H, D, B), jnp.float32),
        scratch_types=[
            pltpu.VMEM((H, bpw), jnp.int32),
            pltpu.VMEM((_NB, 128), jnp.int32),
            pltpu.VMEM((_NB, 128), jnp.int32),
            pltpu.VMEM((_NB, 128, 128), jnp.float32),
            pltpu.VMEM((_NB, D, 128), jnp.float32),
            [pltpu.SemaphoreType.DMA] * _NB,
            [pltpu.SemaphoreType.DMA] * _NB,
        ],
        compiler_params=pltpu.CompilerParams(
            use_tc_tiling_on_sc=True, needs_layout_passes=False),
    )
    def k(tbl_hbm, xt_hbm, out_hbm, idx_all, idx_q, col_q, gbuf, ostage,
          gsem, ssem):
        wid = lax.axis_index("s") * _NC + lax.axis_index("c")
        col0 = wid * bpw
        pltpu.sync_copy(xt_hbm.at[pl.ds(0, h_lo), pl.ds(col0, bpw)],
                        idx_all.at[pl.ds(0, h_lo)])
        pltpu.sync_copy(xt_hbm.at[pl.ds(h_lo, H - h_lo), pl.ds(col0, bpw)],
                        idx_all.at[pl.ds(h_lo, H - h_lo)])

        def prep_and_fire(s, m):
            h = s % H
            bbl = s // H
            for kk in range(8):
                v = idx_all[h, pl.ds(bbl * 128 + kk * _L, _L)]
                idx_q[m, pl.ds(kk * _L, _L)] = lax.shift_right_logical(v, 2)
                col_q[m, pl.ds(kk * _L, _L)] = lax.bitwise_and(v, 3) * D
            pltpu.async_copy(tbl_hbm.at[idx_q.at[m]],
                             gbuf.at[m], gsem[m])

        rowv = [jnp.arange(_L, dtype=jnp.int32) + kk * _L for kk in range(8)]

        for m in range(_NB - 1):
            prep_and_fire(m, m)

        @pl.loop(0, nslab, step=_NB)
        def _(si):
            for m in range(_NB):
                s = si + m

                @pl.when(s + _NB - 1 < nslab)
                def _():
                    prep_and_fire(s + _NB - 1, (m + _NB - 1) % _NB)

                pltpu.make_async_copy(
                    tbl_hbm.at[pl.ds(0, 128)],
                    gbuf.at[m], gsem[m]).wait()

                @pl.when(s >= _NB)
                def _():
                    pltpu.make_async_copy(
                        ostage.at[m],
                        out_hbm.at[0, pl.ds(0, D), pl.ds(0, 128)],
                        ssem[m]).wait()

                colb = [col_q[m, pl.ds(kk * _L, _L)] for kk in range(8)]

                @pl.loop(0, D, step=2)
                def _(d):
                    for u in range(2):
                        drow = lax.bitwise_and(rowv[0] + (d + u), D - 1)
                        for kk in range(8):
                            val = plsc.load_gather(
                                gbuf.at[m], [rowv[kk], colb[kk] + drow])
                            plsc.store_scatter(
                                ostage.at[m], [drow, rowv[kk]], val)

                h = s % H
                bbl = s // H
                pltpu.async_copy(
                    ostage.at[m],
                    out_hbm.at[h, pl.ds(0, D), pl.ds(col0 + bbl * 128, 128)],
                    ssem[m])

        for m in range(_NB):
            pltpu.make_async_copy(
                ostage.at[m],
                out_hbm.at[0, pl.ds(0, D), pl.ds(0, 128)],
                ssem[m]).wait()

    return k


@functools.lru_cache(maxsize=None)
def _make_detile(V, D):
    nt_full = V // 128
    rem = V - nt_full * 128
    base, extra = divmod(nt_full, _NW)
    mesh = plsc.VectorSubcoreMesh(core_axis_name="c", subcore_axis_name="s")

    @functools.partial(
        pl.kernel,
        mesh=mesh,
        out_type=jax.ShapeDtypeStruct((V // 4, 4 * D), jnp.float32),
        scratch_types=[
            pltpu.VMEM((2, D, 128), jnp.float32),
            pltpu.VMEM((2, D, 128), jnp.float32),
            [pltpu.SemaphoreType.DMA] * 2,
            [pltpu.SemaphoreType.DMA] * 2,
        ],
        compiler_params=pltpu.CompilerParams(
            use_tc_tiling_on_sc=True, needs_layout_passes=False),
    )
    def k(tt_hbm, tail_hbm, out_hbm, sbuf, stage, isem, osem):
        wid = lax.axis_index("s") * _NC + lax.axis_index("c")
        t0 = wid * base + jnp.minimum(wid, extra)
        nt = base + jnp.where(wid < extra, 1, 0)

        iota = jnp.arange(_L, dtype=jnp.int32)
        ccol = [iota + 16 * g for g in range(8)]
        srow = [lax.shift_right_logical(c, 2) for c in ccol]
        scb = [lax.bitwise_and(c, 3) * D for c in ccol]

        def fire_in(t, p):
            pltpu.async_copy(
                tt_hbm.at[pl.ds(0, D), pl.ds((t0 + t) * 128, 128)],
                sbuf.at[p], isem[p])

        def transpose(p, ngrp):
            @pl.loop(0, D, step=2)
            def _(d0):
                for u in range(2):
                    drow = lax.bitwise_and(iota + (d0 + u), D - 1)
                    for g in range(ngrp):
                        val = plsc.load_gather(sbuf.at[p], [drow, ccol[g]])
                        plsc.store_scatter(
                            stage.at[p], [srow[g], scb[g] + drow], val)

        fire_in(0, 0)

        @pl.loop(0, nt + (nt % 2), step=2)
        def _(ti):
            for p in (0, 1):
                t = ti + p

                @pl.when(t < nt)
                def _():
                    @pl.when(t + 1 < nt)
                    def _():
                        fire_in(t + 1, 1 - p)

                    pltpu.make_async_copy(
                        tt_hbm.at[pl.ds(0, D), pl.ds(0, 128)],
                        sbuf.at[p], isem[p]).wait()

                    @pl.when(t >= 2)
                    def _():
                        pltpu.make_async_copy(
                            stage.at[p], out_hbm.at[pl.ds(0, D)],
                            osem[p]).wait()

                    transpose(p, 8)
                    pltpu.async_copy(
                        stage.at[p], out_hbm.at[pl.ds((t0 + t) * D, D)],
                        osem[p])

        for p in (0, 1):
            @pl.when(nt >= 2 - p)
            def _():
                pltpu.make_async_copy(
                    stage.at[p], out_hbm.at[pl.ds(0, D)], osem[p]).wait()

        @pl.when(wid == _NW - 1)
        def _():
            pltpu.sync_copy(tail_hbm, out_hbm.at[pl.ds(nt_full * D, rem * D // 128)])

    return k


def kernel(x, table):
    Bx, H = x.shape
    V, D = table.shape
    xt = x.astype(jnp.int32).T
    ttile = table.T
    ntf = (V // 128) * 128
    tail128 = table[ntf:].reshape(-1, 4 * D)
    tbl128 = _make_detile(V, D)(ttile, tail128)
    out3 = _make(V, D, H, Bx)(tbl128, xt)
    return out3.transpose(2, 0, 1)


# speedup vs baseline: 1.4452x; 1.0007x over previous
import functools

import jax
import jax.numpy as jnp
from jax import lax
from jax.experimental import pallas as pl
from jax.experimental.pallas import tpu as pltpu
from jax.experimental.pallas import tpu_sc as plsc

_NC = 2
_NS = 16
_NW = _NC * _NS
_L = 16
_NB = 4


@functools.lru_cache(maxsize=None)
def _make(V, D, H, B):
    assert D == 32 and V % 4 == 0
    bpw = B // _NW
    nslab = H * (bpw // 128)
    assert nslab % _NB == 0
    h_lo = (H // 8) * 8
    mesh = plsc.VectorSubcoreMesh(core_axis_name="c", subcore_axis_name="s")

    @functools.partial(
        pl.kernel,
        mesh=mesh,
        out_type=jax.ShapeDtypeStruct((H, D, B), jnp.float32),
        scratch_types=[
            pltpu.VMEM((H, bpw), jnp.int32),
            pltpu.VMEM((_NB, 128), jnp.int32),
            pltpu.VMEM((_NB, 128), jnp.int32),
            pltpu.VMEM((_NB, 128, 128), jnp.float32),
            pltpu.VMEM((_NB, D, 128), jnp.float32),
            [pltpu.SemaphoreType.DMA] * _NB,
            [pltpu.SemaphoreType.DMA] * _NB,
        ],
        compiler_params=pltpu.CompilerParams(
            use_tc_tiling_on_sc=True, needs_layout_passes=False),
    )
    def k(tbl_hbm, xt_hbm, out_hbm, idx_all, idx_q, col_q, gbuf, ostage,
          gsem, ssem):
        wid = lax.axis_index("s") * _NC + lax.axis_index("c")
        col0 = wid * bpw
        pltpu.sync_copy(xt_hbm.at[pl.ds(0, h_lo), pl.ds(col0, bpw)],
                        idx_all.at[pl.ds(0, h_lo)])
        pltpu.sync_copy(xt_hbm.at[pl.ds(h_lo, H - h_lo), pl.ds(col0, bpw)],
                        idx_all.at[pl.ds(h_lo, H - h_lo)])

        def prep_and_fire(s, m):
            h = s % H
            bbl = s // H
            for kk in range(8):
                v = idx_all[h, pl.ds(bbl * 128 + kk * _L, _L)]
                idx_q[m, pl.ds(kk * _L, _L)] = lax.shift_right_logical(v, 2)
                col_q[m, pl.ds(kk * _L, _L)] = lax.bitwise_and(v, 3) * D
            pltpu.async_copy(tbl_hbm.at[idx_q.at[m]],
                             gbuf.at[m], gsem[m])

        rowv = [jnp.arange(_L, dtype=jnp.int32) + kk * _L for kk in range(8)]

        for m in range(_NB - 1):
            prep_and_fire(m, m)

        @pl.loop(0, nslab, step=_NB)
        def _(si):
            for m in range(_NB):
                s = si + m

                @pl.when(s + _NB - 1 < nslab)
                def _():
                    prep_and_fire(s + _NB - 1, (m + _NB - 1) % _NB)

                pltpu.make_async_copy(
                    tbl_hbm.at[pl.ds(0, 128)],
                    gbuf.at[m], gsem[m]).wait()

                @pl.when(s >= _NB)
                def _():
                    pltpu.make_async_copy(
                        ostage.at[m],
                        out_hbm.at[0, pl.ds(0, D), pl.ds(0, 128)],
                        ssem[m]).wait()

                colb = [col_q[m, pl.ds(kk * _L, _L)] for kk in range(8)]

                @pl.loop(0, D, step=2)
                def _(d):
                    for u in range(2):
                        drow = lax.bitwise_and(rowv[0] + (d + u), D - 1)
                        for kk in range(8):
                            val = plsc.load_gather(
                                gbuf.at[m], [rowv[kk], colb[kk] + drow])
                            plsc.store_scatter(
                                ostage.at[m], [drow, rowv[kk]], val)

                h = s % H
                bbl = s // H
                pltpu.async_copy(
                    ostage.at[m],
                    out_hbm.at[h, pl.ds(0, D), pl.ds(col0 + bbl * 128, 128)],
                    ssem[m])

        for m in range(_NB):
            pltpu.make_async_copy(
                ostage.at[m],
                out_hbm.at[0, pl.ds(0, D), pl.ds(0, 128)],
                ssem[m]).wait()

    return k


@functools.lru_cache(maxsize=None)
def _make_detile(V, D):
    nt_full = V // 128
    rem = V - nt_full * 128
    base, extra = divmod(nt_full, _NW)
    mesh = plsc.VectorSubcoreMesh(core_axis_name="c", subcore_axis_name="s")

    @functools.partial(
        pl.kernel,
        mesh=mesh,
        out_type=jax.ShapeDtypeStruct((V // 4, 4 * D), jnp.float32),
        scratch_types=[
            pltpu.VMEM((2, D, 128), jnp.float32),
            pltpu.VMEM((2, D, 128), jnp.float32),
            [pltpu.SemaphoreType.DMA] * 2,
            [pltpu.SemaphoreType.DMA] * 2,
        ],
        compiler_params=pltpu.CompilerParams(
            use_tc_tiling_on_sc=True, needs_layout_passes=False),
    )
    def k(tt_hbm, tail_hbm, out_hbm, sbuf, stage, isem, osem):
        wid = lax.axis_index("s") * _NC + lax.axis_index("c")
        t0 = wid * base + jnp.minimum(wid, extra)
        nt = base + jnp.where(wid < extra, 1, 0)

        iota = jnp.arange(_L, dtype=jnp.int32)
        ccol = [iota + 16 * g for g in range(8)]
        srow = [lax.shift_right_logical(c, 2) for c in ccol]
        scb = [lax.bitwise_and(c, 3) * D for c in ccol]

        def fire_in(t, p):
            pltpu.async_copy(
                tt_hbm.at[pl.ds(0, D), pl.ds((t0 + t) * 128, 128)],
                sbuf.at[p], isem[p])

        def transpose(p, ngrp):
            @pl.loop(0, D, step=2)
            def _(d0):
                for u in range(2):
                    drow = lax.bitwise_and(iota + (d0 + u), D - 1)
                    for g in range(ngrp):
                        val = plsc.load_gather(sbuf.at[p], [drow, ccol[g]])
                        plsc.store_scatter(
                            stage.at[p], [srow[g], scb[g] + drow], val)

        fire_in(0, 0)

        @pl.loop(0, nt + (nt % 2), step=2)
        def _(ti):
            for p in (0, 1):
                t = ti + p

                @pl.when(t < nt)
                def _():
                    @pl.when(t + 1 < nt)
                    def _():
                        fire_in(t + 1, 1 - p)

                    pltpu.make_async_copy(
                        tt_hbm.at[pl.ds(0, D), pl.ds(0, 128)],
                        sbuf.at[p], isem[p]).wait()

                    @pl.when(t >= 2)
                    def _():
                        pltpu.make_async_copy(
                            stage.at[p], out_hbm.at[pl.ds(0, D)],
                            osem[p]).wait()

                    transpose(p, 8)
                    pltpu.async_copy(
                        stage.at[p], out_hbm.at[pl.ds((t0 + t) * D, D)],
                        osem[p])

        for p in (0, 1):
            @pl.when(nt >= 2 - p)
            def _():
                pltpu.make_async_copy(
                    stage.at[p], out_hbm.at[pl.ds(0, D)], osem[p]).wait()

        @pl.when(wid == _NW - 1)
        def _():
            pltpu.sync_copy(tail_hbm, out_hbm.at[pl.ds(nt_full * D, rem * D // 128)])

    return k


def kernel(x, table):
    Bx, H = x.shape
    V, D = table.shape
    xt = x.astype(jnp.int32).T
    ttile = table.T
    ntf = (V // 128) * 128
    tail128 = table[ntf:].reshape(-1, 4 * D)
    tbl128 = _make_detile(V, D)(ttile, tail128)
    out3 = _make(V, D, H, Bx)(tbl128, xt)
    return out3.transpose(2, 0, 1)
